# baseline (speedup 1.0000x reference)
import functools

import numpy as np

import jax
import jax.numpy as jnp
from jax import lax
from jax.experimental import pallas as pl
from jax.experimental.pallas import tpu as pltpu
from jax.experimental.pallas import tpu_sc as plsc

B = 16384
L = 200
D = 32
V = 1000000
NC = 2
NS = 16
NW = NC * NS
SENT_PER_W = B // NW
C = 8
ROWS = C * L
N_CHUNK = SENT_PER_W // C

P = 1 << 20
E8 = 8
Q8 = P // E8
TB = 1024
GRID = Q8 // TB
LAST_FULL = (V - (E8 - 1) * Q8) // TB
TAIL0 = (E8 - 1) * Q8 + LAST_FULL * TB
NTAIL = P - TAIL0

HI_MASK = np.uint32(0xFFFF0000)


def _fmt_body(*refs):
    (x0, x1, x2, x3, x4, x5, x6, x7a, x7b, out_ref) = refs
    i = pl.program_id(0)
    x7 = jnp.where(i < LAST_FULL, x7a[...], x7b[...])
    xs = [x0[...], x1[...], x2[...], x3[...], x4[...], x5[...], x6[...], x7]
    xlo = jnp.concatenate([x[0:16, :] for x in xs], axis=0)
    xhi = jnp.concatenate([x[16:32, :] for x in xs], axis=0)

    def to_u32(x):
        b = lax.bitcast_convert_type(x.astype(jnp.bfloat16), jnp.uint16)
        return b.astype(jnp.uint32)

    packed = (to_u32(xlo) << 16) | to_u32(xhi)
    out_ref[...] = jnp.transpose(packed)


def _mk_spec(a):
    return pl.BlockSpec((D, TB), lambda i, a=a: (0, a * (Q8 // TB) + i))


def _fmt(embT, tailT):
    in_specs = [_mk_spec(a) for a in range(E8 - 1)]
    in_specs.append(pl.BlockSpec(
        (D, TB),
        lambda i: (0, (E8 - 1) * (Q8 // TB) + jnp.minimum(i, LAST_FULL - 1))))
    in_specs.append(pl.BlockSpec(
        (D, TB),
        lambda i: (0, jnp.clip(i - LAST_FULL, 0, NTAIL // TB - 1))))
    return pl.pallas_call(
        _fmt_body,
        grid=(GRID,),
        in_specs=in_specs,
        out_specs=pl.BlockSpec((TB, 128), lambda i: (i, 0)),
        out_shape=jax.ShapeDtypeStruct((Q8, 128), jnp.uint32),
    )(embT, embT, embT, embT, embT, embT, embT, embT, tailT)


def _body(idx_hbm, table_hbm, out_hbm,
          idx0, idx1, rows0, rows1, out_v, emb0_v, sem0a, sem0b,
          sem1a, sem1b):
    sem0 = (sem0a, sem0b)
    sem1 = (sem1a, sem1b)
    wid = lax.axis_index("s") * NC + lax.axis_index("c")
    sent0 = wid * SENT_PER_W

    pltpu.sync_copy(table_hbm.at[pl.ds(0, 1)], emb0_v)
    e0p = emb0_v[0, pl.ds(0, 16)]
    e0a = plsc.bitcast(e0p & HI_MASK, jnp.float32)
    e0b = plsc.bitcast(e0p << 16, jnp.float32)

    zeros = jnp.zeros((16,), jnp.float32)
    lane_hi = lax.iota(jnp.int32, 16) >= 8

    def start_fetch(g, idx_v, rows_v, sem):
        sent_base = sent0 + g * C
        pltpu.sync_copy(idx_hbm.at[pl.ds(sent_base * L, ROWS)], idx_v)
        h = ROWS // 2
        pltpu.async_copy(table_hbm.at[idx_v.at[pl.ds(0, h)]],
                         rows_v.at[pl.ds(0, h)], sem[0])
        pltpu.async_copy(table_hbm.at[idx_v.at[pl.ds(h, h)]],
                         rows_v.at[pl.ds(h, h)], sem[1])

    def wait_fetch(idx_v, rows_v, sem):
        h = ROWS // 2
        pltpu.make_async_copy(table_hbm.at[idx_v.at[pl.ds(0, h)]],
                              rows_v.at[pl.ds(0, h)], sem[0]).wait()
        pltpu.make_async_copy(table_hbm.at[idx_v.at[pl.ds(h, h)]],
                              rows_v.at[pl.ds(h, h)], sem[1]).wait()

    def compute_chunk(g, idx_v, rows_v):
        sent_base = sent0 + g * C
        for s in range(C):
            row0 = s * L

            def lbody(l, acc):
                a0, a1 = acc
                v = rows_v[row0 + l, pl.ds(0, 16)]
                a0 = a0 + plsc.bitcast(v & HI_MASK, jnp.float32)
                a1 = a1 + plsc.bitcast(v << 16, jnp.float32)
                return (a0, a1)

            a0, a1 = lax.fori_loop(0, L, lbody, (zeros, zeros), unroll=8)

            cnt = jnp.zeros((16,), jnp.int32)
            i_one = jnp.full((16,), 1, jnp.int32)
            i_zero = jnp.zeros((16,), jnp.int32)
            for k in range(12):
                iv = idx_v[pl.ds(row0 + 16 * k, 16)]
                cnt = cnt + jnp.where(iv != 0, i_one, i_zero)
            iv = idx_v[pl.ds(row0 + L - 16, 16)]
            cnt = cnt + jnp.where((iv != 0) & lane_hi, i_one, i_zero)
            vals = [cnt[j] for j in range(16)]
            while len(vals) > 1:
                vals = [vals[k] + vals[k + 1] for k in range(0, len(vals), 2)]
            t = vals[0]
            n_valid = jnp.full((16,), 1.0, jnp.float32) * t.astype(jnp.float32)
            n_pad = jnp.float32(L) - n_valid
            scale = 1.0 / (n_valid + 1e-8)
            out_v[s, pl.ds(0, 16)] = (a0 - n_pad * e0a) * scale
            out_v[s, pl.ds(16, 16)] = (a1 - n_pad * e0b) * scale

        pltpu.sync_copy(out_v, out_hbm.at[pl.ds(sent_base, C)])

    start_fetch(0, idx0, rows0, sem0)

    def pair_body(g2, carry):
        c0 = 2 * g2
        start_fetch(c0 + 1, idx1, rows1, sem1)
        wait_fetch(idx0, rows0, sem0)
        compute_chunk(c0, idx0, rows0)

        @pl.when(c0 + 2 < N_CHUNK)
        def _():
            start_fetch(c0 + 2, idx0, rows0, sem0)

        wait_fetch(idx1, rows1, sem1)
        compute_chunk(c0 + 1, idx1, rows1)
        return carry

    lax.fori_loop(0, N_CHUNK // 2, pair_body, 0)


@jax.jit
def _run(idx_flat, table):
    mesh = plsc.VectorSubcoreMesh(core_axis_name="c", subcore_axis_name="s")
    return pl.kernel(
        _body,
        out_type=jax.ShapeDtypeStruct((B, D), jnp.float32),
        mesh=mesh,
        compiler_params=pltpu.CompilerParams(
            use_tc_tiling_on_sc=False, needs_layout_passes=False),
        scratch_types=[
            pltpu.VMEM((ROWS,), jnp.int32),
            pltpu.VMEM((ROWS,), jnp.int32),
            pltpu.VMEM((ROWS, 16), jnp.uint32),
            pltpu.VMEM((ROWS, 16), jnp.uint32),
            pltpu.VMEM((C, D), jnp.float32),
            pltpu.VMEM((1, 16), jnp.uint32),
            pltpu.SemaphoreType.DMA,
            pltpu.SemaphoreType.DMA,
            pltpu.SemaphoreType.DMA,
            pltpu.SemaphoreType.DMA,
        ],
    )(idx_flat, table)


def kernel(inputs, embeddings):
    t = inputs.astype(jnp.int32)
    idx_flat = (((t & (Q8 - 1)) << 3) | (t >> 17)).reshape(B * L)
    tail = jnp.zeros((NTAIL, D), jnp.float32).at[:V - TAIL0].set(
        embeddings[TAIL0:])
    table = _fmt(embeddings.T, tail.T).reshape(P * 16).reshape(P, 16)
    return _run(idx_flat, table)


# speedup vs baseline: 1.1074x; 1.1074x over previous
import functools

import numpy as np

import jax
import jax.numpy as jnp
from jax import lax
from jax.experimental import pallas as pl
from jax.experimental.pallas import tpu as pltpu
from jax.experimental.pallas import tpu_sc as plsc

B = 16384
L = 200
D = 32
V = 1000000
NC = 2
NS = 16
NW = NC * NS
SENT_PER_W = B // NW
C = 8
ROWS = C * L
N_CHUNK = SENT_PER_W // C

P = 1 << 20
E8 = 8
Q8 = P // E8
TB = 2048
GRID = Q8 // TB
LAST_FULL = (V - (E8 - 1) * Q8) // TB
TAIL0 = (E8 - 1) * Q8 + LAST_FULL * TB
NTAIL = P - TAIL0

HI_MASK = np.uint32(0xFFFF0000)


def _fmt_body(*refs):
    (x0, x1, x2, x3, x4, x5, x6, x7a, x7b, out_ref) = refs
    i = pl.program_id(0)
    x7 = jnp.where(i < LAST_FULL, x7a[...], x7b[...])
    xs = [x0[...], x1[...], x2[...], x3[...], x4[...], x5[...], x6[...], x7]
    xlo = jnp.concatenate([x[0:16, :] for x in xs], axis=0)
    xhi = jnp.concatenate([x[16:32, :] for x in xs], axis=0)

    def to_u32(x):
        b = lax.bitcast_convert_type(x.astype(jnp.bfloat16), jnp.uint16)
        return b.astype(jnp.uint32)

    packed = (to_u32(xlo) << 16) | to_u32(xhi)
    out_ref[...] = jnp.transpose(packed)


def _mk_spec(a):
    return pl.BlockSpec((D, TB), lambda i, a=a: (0, a * (Q8 // TB) + i))


def _fmt(embT, tailT):
    in_specs = [_mk_spec(a) for a in range(E8 - 1)]
    in_specs.append(pl.BlockSpec(
        (D, TB),
        lambda i: (0, (E8 - 1) * (Q8 // TB) + jnp.minimum(i, LAST_FULL - 1))))
    in_specs.append(pl.BlockSpec(
        (D, TB),
        lambda i: (0, jnp.clip(i - LAST_FULL, 0, NTAIL // TB - 1))))
    return pl.pallas_call(
        _fmt_body,
        grid=(GRID,),
        in_specs=in_specs,
        out_specs=pl.BlockSpec((TB, 128), lambda i: (i, 0)),
        out_shape=jax.ShapeDtypeStruct((Q8, 128), jnp.uint32),
    )(embT, embT, embT, embT, embT, embT, embT, embT, tailT)


def _body(idx_hbm, table_hbm, out_hbm,
          idx0, idx1, rows0, rows1, out_v, emb0_v, sem0a, sem0b,
          sem1a, sem1b):
    sem0 = (sem0a, sem0b)
    sem1 = (sem1a, sem1b)
    wid = lax.axis_index("s") * NC + lax.axis_index("c")
    sent0 = wid * SENT_PER_W

    pltpu.sync_copy(table_hbm.at[pl.ds(0, 1)], emb0_v)
    e0p = emb0_v[0, pl.ds(0, 16)]
    e0a = plsc.bitcast(e0p & HI_MASK, jnp.float32)
    e0b = plsc.bitcast(e0p << 16, jnp.float32)

    zeros = jnp.zeros((16,), jnp.float32)
    lane_hi = lax.iota(jnp.int32, 16) >= 8

    def start_fetch(g, idx_v, rows_v, sem):
        sent_base = sent0 + g * C
        pltpu.sync_copy(idx_hbm.at[pl.ds(sent_base * L, ROWS)], idx_v)

        def tbody(k, carry):
            t = idx_v[pl.ds(16 * k, 16)]
            idx_v[pl.ds(16 * k, 16)] = ((t & (Q8 - 1)) << 3) | (t >> 17)
            return carry

        lax.fori_loop(0, ROWS // 16, tbody, 0, unroll=4)
        h = ROWS // 2
        pltpu.async_copy(table_hbm.at[idx_v.at[pl.ds(0, h)]],
                         rows_v.at[pl.ds(0, h)], sem[0])
        pltpu.async_copy(table_hbm.at[idx_v.at[pl.ds(h, h)]],
                         rows_v.at[pl.ds(h, h)], sem[1])

    def wait_fetch(idx_v, rows_v, sem):
        h = ROWS // 2
        pltpu.make_async_copy(table_hbm.at[idx_v.at[pl.ds(0, h)]],
                              rows_v.at[pl.ds(0, h)], sem[0]).wait()
        pltpu.make_async_copy(table_hbm.at[idx_v.at[pl.ds(h, h)]],
                              rows_v.at[pl.ds(h, h)], sem[1]).wait()

    def compute_chunk(g, idx_v, rows_v):
        sent_base = sent0 + g * C
        for s in range(C):
            row0 = s * L

            def lbody(l, acc):
                a0, a1 = acc
                v = rows_v[row0 + l, pl.ds(0, 16)]
                a0 = a0 + plsc.bitcast(v & HI_MASK, jnp.float32)
                a1 = a1 + plsc.bitcast(v << 16, jnp.float32)
                return (a0, a1)

            a0, a1 = lax.fori_loop(0, L, lbody, (zeros, zeros), unroll=8)

            cnt = jnp.zeros((16,), jnp.int32)
            i_one = jnp.full((16,), 1, jnp.int32)
            i_zero = jnp.zeros((16,), jnp.int32)
            for k in range(12):
                iv = idx_v[pl.ds(row0 + 16 * k, 16)]
                cnt = cnt + jnp.where(iv != 0, i_one, i_zero)
            iv = idx_v[pl.ds(row0 + L - 16, 16)]
            cnt = cnt + jnp.where((iv != 0) & lane_hi, i_one, i_zero)
            t = cnt[0]
            for j in range(1, 16):
                t = t + cnt[j]
            n_valid = jnp.full((16,), 1.0, jnp.float32) * t.astype(jnp.float32)
            n_pad = jnp.float32(L) - n_valid
            scale = 1.0 / (n_valid + 1e-8)
            out_v[s, pl.ds(0, 16)] = (a0 - n_pad * e0a) * scale
            out_v[s, pl.ds(16, 16)] = (a1 - n_pad * e0b) * scale

        pltpu.sync_copy(out_v, out_hbm.at[pl.ds(sent_base, C)])

    start_fetch(0, idx0, rows0, sem0)

    def pair_body(g2, carry):
        c0 = 2 * g2
        start_fetch(c0 + 1, idx1, rows1, sem1)
        wait_fetch(idx0, rows0, sem0)
        compute_chunk(c0, idx0, rows0)

        @pl.when(c0 + 2 < N_CHUNK)
        def _():
            start_fetch(c0 + 2, idx0, rows0, sem0)

        wait_fetch(idx1, rows1, sem1)
        compute_chunk(c0 + 1, idx1, rows1)
        return carry

    lax.fori_loop(0, N_CHUNK // 2, pair_body, 0)


@jax.jit
def _run(idx_flat, table):
    mesh = plsc.VectorSubcoreMesh(core_axis_name="c", subcore_axis_name="s")
    return pl.kernel(
        _body,
        out_type=jax.ShapeDtypeStruct((B, D), jnp.float32),
        mesh=mesh,
        compiler_params=pltpu.CompilerParams(
            use_tc_tiling_on_sc=False, needs_layout_passes=False),
        scratch_types=[
            pltpu.VMEM((ROWS,), jnp.int32),
            pltpu.VMEM((ROWS,), jnp.int32),
            pltpu.VMEM((ROWS, 16), jnp.uint32),
            pltpu.VMEM((ROWS, 16), jnp.uint32),
            pltpu.VMEM((C, D), jnp.float32),
            pltpu.VMEM((1, 16), jnp.uint32),
            pltpu.SemaphoreType.DMA,
            pltpu.SemaphoreType.DMA,
            pltpu.SemaphoreType.DMA,
            pltpu.SemaphoreType.DMA,
        ],
    )(idx_flat, table)


def kernel(inputs, embeddings):
    idx_flat = inputs.astype(jnp.int32).reshape(B * L)
    tail = jnp.zeros((NTAIL, D), jnp.float32).at[:V - TAIL0].set(
        embeddings[TAIL0:])
    table = _fmt(embeddings.T, tail.T).reshape(P * 16).reshape(P, 16)
    return _run(idx_flat, table)
